# baseline (speedup 1.0000x reference)
import functools

import jax
import jax.numpy as jnp
from jax import lax
from jax.experimental import pallas as pl
from jax.experimental.pallas import tpu as pltpu
from jax.experimental.pallas import tpu_sc as plsc

_BT = 32768
_D = 64
_NW = 32
_ROWS_W = _BT // _NW
_CHUNK = 512
_NCHUNK = _ROWS_W // _CHUNK
_IM = 128
_JPC = _CHUNK // _IM
_LMBDA = 0.01


def _sc_gather_score(entity_emb, relation_emb, h_idx, t_idx, r_idx):
    mesh = plsc.VectorSubcoreMesh(core_axis_name="c", subcore_axis_name="s")

    @functools.partial(
        pl.kernel,
        mesh=mesh,
        compiler_params=pltpu.CompilerParams(
            needs_layout_passes=False, use_tc_tiling_on_sc=False),
        out_type=[
            jax.ShapeDtypeStruct((_BT,), jnp.float32),
            jax.ShapeDtypeStruct((_NW, 16), jnp.float32),
        ],
        scratch_types=[
            pltpu.VMEM((_JPC, _IM), jnp.int32),
            pltpu.VMEM((_JPC, _IM), jnp.int32),
            pltpu.VMEM((_JPC, _IM), jnp.int32),
            pltpu.VMEM((_CHUNK, _D), jnp.float32),
            pltpu.VMEM((_CHUNK, _D), jnp.float32),
            pltpu.VMEM((_CHUNK, _D), jnp.float32),
            pltpu.VMEM((_CHUNK,), jnp.float32),
            pltpu.VMEM((256,), jnp.float32),
            pltpu.VMEM((16,), jnp.float32),
            pltpu.SemaphoreType.DMA,
        ],
    )
    def k(ent_hbm, rel_hbm, hidx_hbm, tidx_hbm, ridx_hbm,
          scores_out, sumsq_out,
          hidx_v, tidx_v, ridx_v, hrows, trows, rrows, scores_v, pbuf, sq_v,
          sem):
        wid = lax.axis_index("s") * 2 + lax.axis_index("c")
        lane = lax.iota(jnp.int32, 16)
        sq = jnp.zeros((16,), jnp.float32)
        for c in range(_NCHUNK):
            row0 = wid * (_ROWS_W // _IM) + c * _JPC
            pltpu.sync_copy(hidx_hbm.at[pl.ds(row0, _JPC)], hidx_v)
            pltpu.sync_copy(tidx_hbm.at[pl.ds(row0, _JPC)], tidx_v)
            pltpu.sync_copy(ridx_hbm.at[pl.ds(row0, _JPC)], ridx_v)
            copies = []
            for j in range(_JPC):
                dst = pl.ds(j * _IM, _IM)
                copies.append(pltpu.async_copy(
                    ent_hbm.at[hidx_v.at[j]], hrows.at[dst], sem))
                copies.append(pltpu.async_copy(
                    ent_hbm.at[tidx_v.at[j]], trows.at[dst], sem))
                copies.append(pltpu.async_copy(
                    rel_hbm.at[ridx_v.at[j]], rrows.at[dst], sem))
            for cp in copies:
                cp.wait()

            def outer(bi, sq):
                def rowfn(i, sq):
                    row = bi * 16 + i
                    hr = hrows.at[row]
                    tr = trows.at[row]
                    rr = rrows.at[row]
                    p = jnp.zeros((16,), jnp.float32)
                    for g in range(_D // 16):
                        seg = pl.ds(g * 16, 16)
                        hv = hr[seg]
                        tv = tr[seg]
                        rv = rr[seg]
                        p = p + hv * tv * rv
                        sq = sq + hv * hv + tv * tv + rv * rv
                    pbuf[pl.ds(i * 16, 16)] = p
                    return sq

                sq = lax.fori_loop(0, 16, rowfn, sq)

                def colfn(j, acc):
                    return acc + plsc.load_gather(pbuf, [lane * 16 + j])

                acc = lax.fori_loop(0, 16, colfn,
                                    jnp.zeros((16,), jnp.float32))
                scores_v[pl.ds(bi * 16, 16)] = acc
                return sq

            sq = lax.fori_loop(0, _CHUNK // 16, outer, sq)
            pltpu.sync_copy(
                scores_v,
                scores_out.at[pl.ds(wid * _ROWS_W + c * _CHUNK, _CHUNK)])
        sq_v[...] = sq
        pltpu.sync_copy(sq_v, sumsq_out.at[wid])

    return k(entity_emb, relation_emb, h_idx, t_idx, r_idx)


def _tc_finalize(scores2d, sumsq2d):
    nrow = scores2d.shape[0]

    def body(s_ref, q_ref, o_ref):
        s = s_ref[...]
        row = lax.broadcasted_iota(jnp.int32, s.shape, 0)
        y = jnp.where(row < nrow // 2, 1.0, -1.0).astype(jnp.float32)
        z = -s * y
        sp = jnp.maximum(z, 0.0) + jnp.log1p(jnp.exp(-jnp.abs(z)))
        regul = jnp.sum(q_ref[...]) / float(_BT * _D)
        o_ref[0, 0] = jnp.sum(sp) / float(_BT) + _LMBDA * regul

    out = pl.pallas_call(
        body,
        out_shape=jax.ShapeDtypeStruct((1, 1), jnp.float32),
        out_specs=pl.BlockSpec(memory_space=pltpu.SMEM),
    )(scores2d, sumsq2d)
    return out


def kernel(pos_h, pos_r, pos_t, neg_h, neg_r, neg_t, entity_emb, relation_emb):
    h_idx = jnp.concatenate([pos_h, neg_h]).reshape(_BT // _IM, _IM)
    t_idx = jnp.concatenate([pos_t, neg_t]).reshape(_BT // _IM, _IM)
    r_idx = jnp.concatenate([pos_r[:, 0], neg_r[:, 0]]).reshape(_BT // _IM, _IM)
    scores, sumsq = _sc_gather_score(
        entity_emb, relation_emb, h_idx, t_idx, r_idx)
    out = _tc_finalize(scores.reshape(_BT // _IM, _IM),
                       sumsq.reshape(_NW * 16 // _IM, _IM))
    return out.reshape(())
